# baseline (speedup 1.0000x reference)
import functools

import jax
import jax.numpy as jnp
from jax import lax
from jax.experimental import pallas as pl
from jax.experimental.pallas import tpu as pltpu
from jax.experimental.pallas import tpu_sc as plsc

N = 10000
NPAD = 10240
F = 128
NCLASS = 64
THR = 0.1
NC = 2
NS = 16
NW = NC * NS
C = 128
EPT = 10240
NCH = EPT // C
EPAD = NW * EPT
SL = NPAD // NS
f32 = jnp.float32
i32 = jnp.int32

_mesh = plsc.VectorSubcoreMesh(core_axis_name="c", subcore_axis_name="s")


def _make_sim_pass(has_prev):
    out_type = (jax.ShapeDtypeStruct((NW, NCH, C), f32),
                jax.ShapeDtypeStruct((NW, NPAD), f32))
    scratch = [
        pltpu.VMEM((NCH, C), i32),
        pltpu.VMEM((NCH, C), i32),
        pltpu.VMEM((C, F), f32),
        pltpu.VMEM((C, F), f32),
        pltpu.VMEM((C, F), f32),
        pltpu.VMEM((C, F), f32),
        pltpu.VMEM((C,), f32),
        pltpu.VMEM((NPAD,), f32),
        pltpu.SemaphoreType.DMA,
        pltpu.SemaphoreType.DMA,
    ]
    if has_prev:
        scratch.insert(2, pltpu.VMEM((NCH, C), f32))

    def body(*refs):
        if has_prev:
            (feat_hbm, rowp_hbm, colp_hbm, prev_hbm, val_hbm, deg_hbm,
             rowv, colv, prevv, rbuf, cbuf, rbuf2, cbuf2, valb, degl,
             sem1, sem2) = refs
        else:
            (feat_hbm, rowp_hbm, colp_hbm, val_hbm, deg_hbm,
             rowv, colv, rbuf, cbuf, rbuf2, cbuf2, valb, degl,
             sem1, sem2) = refs
        cid = lax.axis_index("c")
        sid = lax.axis_index("s")
        wid = sid * NC + cid
        pltpu.sync_copy(rowp_hbm.at[wid], rowv)
        pltpu.sync_copy(colp_hbm.at[wid], colv)
        if has_prev:
            pltpu.sync_copy(prev_hbm.at[wid], prevv)

        zero16 = jnp.zeros((16,), f32)

        def zbody(i, carry):
            degl[pl.ds(i * 16, 16)] = zero16
            return carry
        lax.fori_loop(0, NPAD // 16, zbody, 0)

        iota = lax.iota(i32, 16)
        zv = jnp.zeros((16,), f32)
        z0 = jnp.zeros((16,), i32)

        def compute(j, rbuf_, cbuf_):
            for g in range(C // 16):
                rows16 = iota + (g * 16)

                def dot(i, accs):
                    a0, a1, a2, a3 = accs
                    kv = jnp.full((16,), i, i32) + iota
                    k0 = kv & (F - 1)
                    k1 = (kv + 1) & (F - 1)
                    k2 = (kv + 2) & (F - 1)
                    k3 = (kv + 3) & (F - 1)
                    x0 = plsc.load_gather(rbuf_, [rows16, k0])
                    y0 = plsc.load_gather(cbuf_, [rows16, k0])
                    x1 = plsc.load_gather(rbuf_, [rows16, k1])
                    y1 = plsc.load_gather(cbuf_, [rows16, k1])
                    x2 = plsc.load_gather(rbuf_, [rows16, k2])
                    y2 = plsc.load_gather(cbuf_, [rows16, k2])
                    x3 = plsc.load_gather(rbuf_, [rows16, k3])
                    y3 = plsc.load_gather(cbuf_, [rows16, k3])
                    return (a0 + x0 * y0, a1 + x1 * y1,
                            a2 + x2 * y2, a3 + x3 * y3)
                a0, a1, a2, a3 = plsc.parallel_loop(
                    0, F, 4, unroll=4, carry=(zv, zv, zv, zv))(dot)
                sims = (a0 + a1) + (a2 + a3)
                rv = rowv[j, pl.ds(g * 16, 16)]
                cv = colv[j, pl.ds(g * 16, 16)]
                m = (sims >= THR) & (rv != cv)
                if has_prev:
                    m = m & (prevv[j, pl.ds(g * 16, 16)] > 0.0)
                val = jnp.where(m, sims, 0.0)
                valb[pl.ds(g * 16, 16)] = val
                plsc.addupdate_scatter(degl, [rv], val)
            pltpu.sync_copy(valb, val_hbm.at[wid, j])

        def chunk2(jj, carry):
            j = jj * 2

            @pl.when(jj == 0)
            def _():
                pltpu.async_copy(feat_hbm.at[rowv.at[j]], rbuf, sem1)
                pltpu.async_copy(feat_hbm.at[colv.at[j]], cbuf, sem1)
            pltpu.make_async_copy(feat_hbm.at[rowv.at[j]], rbuf, sem1).wait()
            pltpu.make_async_copy(feat_hbm.at[colv.at[j]], cbuf, sem1).wait()
            pltpu.async_copy(feat_hbm.at[rowv.at[j + 1]], rbuf2, sem2)
            pltpu.async_copy(feat_hbm.at[colv.at[j + 1]], cbuf2, sem2)
            compute(j, rbuf, cbuf)
            pltpu.make_async_copy(
                feat_hbm.at[rowv.at[j + 1]], rbuf2, sem2).wait()
            pltpu.make_async_copy(
                feat_hbm.at[colv.at[j + 1]], cbuf2, sem2).wait()

            @pl.when(jj < NCH // 2 - 1)
            def _():
                pltpu.async_copy(feat_hbm.at[rowv.at[j + 2]], rbuf, sem1)
                pltpu.async_copy(feat_hbm.at[colv.at[j + 2]], cbuf, sem1)
            compute(j + 1, rbuf2, cbuf2)
            return carry
        lax.fori_loop(0, NCH // 2, chunk2, 0)
        pltpu.sync_copy(degl, deg_hbm.at[wid])

    return pl.kernel(body, out_type=out_type, mesh=_mesh,
                     compiler_params=pltpu.CompilerParams(
                         needs_layout_passes=False),
                     scratch_types=scratch)


def _make_agg_pass(D, CB):
    NCB = EPT // CB
    T = NCB // 4
    DH = D // 2
    out_type = jax.ShapeDtypeStruct((NC, NPAD, D), f32)
    scratch = [
        pltpu.VMEM((3, CB), i32),
        pltpu.VMEM((3, CB), i32),
        pltpu.VMEM((3, CB), i32),
        pltpu.VMEM((3, CB), i32),
        pltpu.VMEM((CB,), i32),
        pltpu.VMEM((CB,), i32),
        pltpu.VMEM((CB,), i32),
        pltpu.VMEM((CB,), i32),
        pltpu.VMEM((CB, DH), i32),
        pltpu.VMEM((CB, DH), i32),
        pltpu.VMEM((CB, DH), i32),
        pltpu.VMEM((CB, DH), i32),
        pltpu.VMEM((CB, D), f32),
        pltpu.VMEM((CB, D), f32),
        pltpu.VMEM((CB, D), f32),
        pltpu.VMEM((CB, D), f32),
        pltpu.VMEM((CB + 16,), f32),
        pltpu.VMEM_SHARED((NPAD, D), f32),
        pltpu.SemaphoreType.DMA,
        pltpu.SemaphoreType.DMA,
        pltpu.SemaphoreType.DMA,
    ]

    def body(feat_hbm, ivp_hbm, acc_hbm, iv0, iv1, iv2, iv3,
             rowb0, rowb1, rowb2, rowb3, gi0, gi1, gi2, gi3,
             sbuf0, sbuf1, sbuf2, sbuf3, wbuf, acc_sh,
             semi, semg, sems):
        cid = lax.axis_index("c")
        sid = lax.axis_index("s")
        wid = sid * NC + cid

        zero16 = jnp.zeros((16,), f32)
        himask = jnp.full((16,), -65536, i32)

        def zrow(r, carry):
            for k in range(D // 16):
                sbuf0[r, pl.ds(k * 16, 16)] = zero16
            return carry
        lax.fori_loop(0, CB, zrow, 0)

        def zacc(t, carry):
            pltpu.sync_copy(sbuf0, acc_sh.at[pl.ds(sid * SL + t * CB, CB)])
            return carry
        lax.fori_loop(0, SL // CB, zacc, 0)
        plsc.subcore_barrier()

        def compute_scale(iv, rowb, gi, sbuf):
            for g in range(CB // 16):
                sl16 = pl.ds(g * 16, 16)
                wbuf[sl16] = plsc.bitcast(iv[2, sl16], f32)
                rowb[sl16] = iv[0, sl16]

            @plsc.parallel_loop(0, CB, 1, unroll=4)
            def scale(e):
                w16 = wbuf[pl.ds(e, 16)]
                wv = jnp.full((16,), w16[0], f32)
                for k in range(D // 32):
                    wi = gi[e, pl.ds(k * 16, 16)]
                    lo = plsc.bitcast(wi << 16, f32)
                    hi = plsc.bitcast(wi & himask, f32)
                    sbuf[e, pl.ds(k * 32, 16)] = lo * wv
                    sbuf[e, pl.ds(k * 32 + 16, 16)] = hi * wv

        def wait_iv(j, iv):
            pltpu.make_async_copy(ivp_hbm.at[wid, j], iv, semi).wait()

        def wait_gather(iv, gi):
            pltpu.make_async_copy(feat_hbm.at[iv.at[1]], gi, semg).wait()

        def wait_scatter(sbuf, rowb):
            pltpu.make_async_copy(sbuf, acc_sh.at[rowb], sems).wait()

        ivs = (iv0, iv1, iv2, iv3)
        rowbs = (rowb0, rowb1, rowb2, rowb3)
        gis = (gi0, gi1, gi2, gi3)
        sbufs = (sbuf0, sbuf1, sbuf2, sbuf3)

        def quad(t, carry):
            j = 4 * t

            @pl.when(t == 0)
            def _():
                pltpu.sync_copy(ivp_hbm.at[wid, 0 * t], iv0)
                pltpu.async_copy(feat_hbm.at[iv0.at[1]], gi0, semg)
                pltpu.async_copy(ivp_hbm.at[wid, 0 * t + 1], iv1, semi)

            for m in range(4):
                k = j + m
                cur, nxt, nn = m, (m + 1) % 4, (m + 2) % 4
                wait_gather(ivs[cur], gis[cur])
                if m < 3:
                    wait_iv(k + 1, ivs[nxt])

                    @pl.when(jnp.logical_or(t > 0, m >= 3))
                    def _():
                        wait_scatter(sbufs[nxt], rowbs[nxt])
                    pltpu.async_copy(feat_hbm.at[ivs[nxt].at[1]],
                                     gis[nxt], semg)
                else:
                    @pl.when(t < T - 1)
                    def _():
                        wait_iv(k + 1, ivs[nxt])
                    wait_scatter(sbufs[nxt], rowbs[nxt])

                    @pl.when(t < T - 1)
                    def _():
                        pltpu.async_copy(feat_hbm.at[ivs[nxt].at[1]],
                                         gis[nxt], semg)
                compute_scale(ivs[cur], rowbs[cur], gis[cur], sbufs[cur])
                pltpu.async_copy(sbufs[cur], acc_sh.at[rowbs[cur]],
                                 sems, add=True)
                if m < 2:
                    pltpu.async_copy(ivp_hbm.at[wid, k + 2], ivs[nn], semi)
                else:
                    @pl.when(t < T - 1)
                    def _():
                        pltpu.async_copy(ivp_hbm.at[wid, k + 2],
                                         ivs[nn], semi)
            return carry
        lax.fori_loop(0, T, quad, 0)
        wait_scatter(sbuf1, rowb1)
        wait_scatter(sbuf2, rowb2)
        wait_scatter(sbuf3, rowb3)
        plsc.subcore_barrier()

        def dump(t, carry):
            pltpu.sync_copy(acc_sh.at[pl.ds(sid * SL + t * CB, CB)], sbuf0)
            pltpu.sync_copy(sbuf0, acc_hbm.at[cid, pl.ds(sid * SL + t * CB, CB)])
            return carry
        lax.fori_loop(0, SL // CB, dump, 0)

    return pl.kernel(body, out_type=out_type, mesh=_mesh,
                     compiler_params=pltpu.CompilerParams(
                         needs_layout_passes=False,
                         use_tc_tiling_on_sc=False),
                     scratch_types=scratch)


_sim_pass1 = _make_sim_pass(False)
_sim_pass2 = _make_sim_pass(True)
CB1 = 32
CB2 = 64
_agg_pass1 = _make_agg_pass(F, CB1)
_agg_pass2 = _make_agg_pass(NCLASS, CB2)


def _tc1_body(x_ref, xn_ref):
    x = x_ref[...]
    nr = jnp.sqrt(jnp.sum(x * x, axis=1, keepdims=True))
    xn_ref[...] = x / jnp.maximum(nr, 1e-8)


def _tc1(x):
    return pl.pallas_call(
        _tc1_body,
        out_shape=jax.ShapeDtypeStruct((N, F), f32),
    )(x)


def _pack_tc(z):
    u = lax.bitcast_convert_type(z.astype(jnp.bfloat16), jnp.uint16)
    words = []
    for k in range(z.shape[1] // 32):
        lo = u[:, k * 32:k * 32 + 16].astype(jnp.uint32)
        hi = u[:, k * 32 + 16:k * 32 + 32].astype(jnp.uint32)
        words.append(lo | (hi << 16))
    return lax.bitcast_convert_type(jnp.concatenate(words, axis=1), i32)


def _tc2_body(degp_ref, x_ref, dinv_ref, z1p_ref):
    deg = jnp.sum(degp_ref[...], axis=0)[:, None] + 1.0
    dinv = lax.rsqrt(deg)
    dinv_ref[...] = dinv
    z1p_ref[...] = _pack_tc(dinv[:N] * x_ref[...])


def _tc2(degp, x):
    return pl.pallas_call(
        _tc2_body,
        out_shape=(jax.ShapeDtypeStruct((NPAD, 1), f32),
                   jax.ShapeDtypeStruct((N, F // 2), i32)),
    )(degp, x)


def _tc3_body(accp_ref, x_ref, dinv_ref, W1_ref, b1_ref, hn_ref, nr2c_ref):
    dinv = dinv_ref[...][:N]
    acc = accp_ref[0, :N] + accp_ref[1, :N]
    pre = dinv * acc + (dinv * dinv) * x_ref[...]
    h = jnp.maximum(jnp.dot(pre, W1_ref[...],
                            preferred_element_type=f32) + b1_ref[...], 0.0)
    nr2 = jnp.sqrt(jnp.sum(h * h, axis=1, keepdims=True))
    nr2c = jnp.maximum(nr2, 1e-8)
    hn_ref[...] = h / nr2c
    nr2c_ref[...] = jnp.concatenate(
        [nr2c, jnp.ones((NPAD - N, 1), f32)], axis=0)


def _tc3(accp, x, dinv1, W1, b1):
    return pl.pallas_call(
        _tc3_body,
        out_shape=(jax.ShapeDtypeStruct((N, F), f32),
                   jax.ShapeDtypeStruct((NPAD, 1), f32)),
    )(accp, x, dinv1, W1, b1)


def _tc4_body(degp_ref, nr2c_ref, hn_ref, W2_ref, dinv_ref, z2_ref, z2p_ref):
    deg = jnp.sum(degp_ref[...], axis=0)[:, None] + 1.0
    dinv = lax.rsqrt(deg)
    dinv_ref[...] = dinv
    scale = (dinv * nr2c_ref[...])[:N]
    z2 = jnp.dot(scale * hn_ref[...], W2_ref[...],
                 preferred_element_type=f32)
    z2_ref[...] = z2
    z2p_ref[...] = _pack_tc(z2)


def _tc4(degp, nr2c, hn, W2):
    return pl.pallas_call(
        _tc4_body,
        out_shape=(jax.ShapeDtypeStruct((NPAD, 1), f32),
                   jax.ShapeDtypeStruct((N, NCLASS), f32),
                   jax.ShapeDtypeStruct((N, NCLASS // 2), i32)),
    )(degp, nr2c, hn, W2)


def _tc5_body(accp_ref, z2_ref, dinv_ref, b2_ref, out_ref):
    acc = accp_ref[0, :N] + accp_ref[1, :N] + z2_ref[...]
    out_ref[...] = dinv_ref[...][:N] * acc + b2_ref[...]


def _tc5(accp, z2, dinv2, b2):
    return pl.pallas_call(
        _tc5_body,
        out_shape=jax.ShapeDtypeStruct((N, NCLASS), f32),
    )(accp, z2, dinv2, b2)


def _make_ivp(rowf, colf, val, CB):
    NCB = EPT // CB
    r = rowf.reshape(NW, NCB, 1, CB)
    c = colf.reshape(NW, NCB, 1, CB)
    v = lax.bitcast_convert_type(val, i32).reshape(NW, NCB, 1, CB)
    return jnp.concatenate([r, c, v], axis=2)


def kernel(x, adj, W1, b1, W2, b2):
    E = adj.shape[1]
    pad = EPAD - E
    row = adj[0]
    col = adj[1]
    zpad = jnp.arange(pad, dtype=i32) % N
    rowf = jnp.concatenate([row, zpad])
    colf = jnp.concatenate([col, zpad])
    rowp = rowf.reshape(NW, NCH, C)
    colp = colf.reshape(NW, NCH, C)

    xn = _tc1(x)
    val1, deg1p = _sim_pass1(xn, rowp, colp)
    dinv1, z1p = _tc2(deg1p, x)
    acc1p = _agg_pass1(z1p, _make_ivp(rowf, colf, val1, CB1))
    hn, nr2c = _tc3(acc1p, x, dinv1, W1, b1)
    val2, deg2p = _sim_pass2(hn, rowp, colp, val1)
    dinv2, z2, z2p = _tc4(deg2p, nr2c, hn, W2)
    acc2p = _agg_pass2(z2p, _make_ivp(rowf, colf, val2, CB2))
    return _tc5(acc2p, z2, dinv2, b2)


# speedup vs baseline: 1.1787x; 1.1787x over previous
import functools

import jax
import jax.numpy as jnp
from jax import lax
from jax.experimental import pallas as pl
from jax.experimental.pallas import tpu as pltpu
from jax.experimental.pallas import tpu_sc as plsc

N = 10000
NPAD = 10240
F = 128
NCLASS = 64
THR = 0.1
NC = 2
NS = 16
NW = NC * NS
C = 128
EPT = 10752
NCH = EPT // C
EPAD = NW * EPT
SL = NPAD // NS
f32 = jnp.float32
i32 = jnp.int32

_mesh = plsc.VectorSubcoreMesh(core_axis_name="c", subcore_axis_name="s")


def _make_sim_pass(has_prev):
    out_type = (jax.ShapeDtypeStruct((NW, NCH, C), f32),
                jax.ShapeDtypeStruct((NW, NPAD), f32))
    scratch = [
        pltpu.VMEM((NCH, C), i32),
        pltpu.VMEM((NCH, C), i32),
        pltpu.VMEM((C, F), f32),
        pltpu.VMEM((C, F), f32),
        pltpu.VMEM((C, F), f32),
        pltpu.VMEM((C, F), f32),
        pltpu.VMEM((C,), f32),
        pltpu.VMEM((NPAD,), f32),
        pltpu.SemaphoreType.DMA,
        pltpu.SemaphoreType.DMA,
    ]
    if has_prev:
        scratch.insert(2, pltpu.VMEM((NCH, C), f32))

    def body(*refs):
        if has_prev:
            (feat_hbm, rowp_hbm, colp_hbm, prev_hbm, val_hbm, deg_hbm,
             rowv, colv, prevv, rbuf, cbuf, rbuf2, cbuf2, valb, degl,
             sem1, sem2) = refs
        else:
            (feat_hbm, rowp_hbm, colp_hbm, val_hbm, deg_hbm,
             rowv, colv, rbuf, cbuf, rbuf2, cbuf2, valb, degl,
             sem1, sem2) = refs
        cid = lax.axis_index("c")
        sid = lax.axis_index("s")
        wid = sid * NC + cid
        pltpu.sync_copy(rowp_hbm.at[wid], rowv)
        pltpu.sync_copy(colp_hbm.at[wid], colv)
        if has_prev:
            pltpu.sync_copy(prev_hbm.at[wid], prevv)

        zero16 = jnp.zeros((16,), f32)

        def zbody(i, carry):
            degl[pl.ds(i * 16, 16)] = zero16
            return carry
        lax.fori_loop(0, NPAD // 16, zbody, 0)

        iota = lax.iota(i32, 16)
        zv = jnp.zeros((16,), f32)
        z0 = jnp.zeros((16,), i32)

        def compute(j, rbuf_, cbuf_):
            for g in range(C // 16):
                rows16 = iota + (g * 16)

                def dot(i, accs):
                    a0, a1, a2, a3 = accs
                    kv = jnp.full((16,), i, i32) + iota
                    k0 = kv & (F - 1)
                    k1 = (kv + 1) & (F - 1)
                    k2 = (kv + 2) & (F - 1)
                    k3 = (kv + 3) & (F - 1)
                    x0 = plsc.load_gather(rbuf_, [rows16, k0])
                    y0 = plsc.load_gather(cbuf_, [rows16, k0])
                    x1 = plsc.load_gather(rbuf_, [rows16, k1])
                    y1 = plsc.load_gather(cbuf_, [rows16, k1])
                    x2 = plsc.load_gather(rbuf_, [rows16, k2])
                    y2 = plsc.load_gather(cbuf_, [rows16, k2])
                    x3 = plsc.load_gather(rbuf_, [rows16, k3])
                    y3 = plsc.load_gather(cbuf_, [rows16, k3])
                    return (a0 + x0 * y0, a1 + x1 * y1,
                            a2 + x2 * y2, a3 + x3 * y3)
                a0, a1, a2, a3 = plsc.parallel_loop(
                    0, F, 4, unroll=4, carry=(zv, zv, zv, zv))(dot)
                sims = (a0 + a1) + (a2 + a3)
                rv = rowv[j, pl.ds(g * 16, 16)]
                cv = colv[j, pl.ds(g * 16, 16)]
                m = (sims >= THR) & (rv != cv)
                if has_prev:
                    m = m & (prevv[j, pl.ds(g * 16, 16)] > 0.0)
                val = jnp.where(m, sims, 0.0)
                valb[pl.ds(g * 16, 16)] = val
                plsc.addupdate_scatter(degl, [rv], val)
            pltpu.sync_copy(valb, val_hbm.at[wid, j])

        def chunk2(jj, carry):
            j = jj * 2

            @pl.when(jj == 0)
            def _():
                pltpu.async_copy(feat_hbm.at[rowv.at[j]], rbuf, sem1)
                pltpu.async_copy(feat_hbm.at[colv.at[j]], cbuf, sem1)
            pltpu.make_async_copy(feat_hbm.at[rowv.at[j]], rbuf, sem1).wait()
            pltpu.make_async_copy(feat_hbm.at[colv.at[j]], cbuf, sem1).wait()
            pltpu.async_copy(feat_hbm.at[rowv.at[j + 1]], rbuf2, sem2)
            pltpu.async_copy(feat_hbm.at[colv.at[j + 1]], cbuf2, sem2)
            compute(j, rbuf, cbuf)
            pltpu.make_async_copy(
                feat_hbm.at[rowv.at[j + 1]], rbuf2, sem2).wait()
            pltpu.make_async_copy(
                feat_hbm.at[colv.at[j + 1]], cbuf2, sem2).wait()

            @pl.when(jj < NCH // 2 - 1)
            def _():
                pltpu.async_copy(feat_hbm.at[rowv.at[j + 2]], rbuf, sem1)
                pltpu.async_copy(feat_hbm.at[colv.at[j + 2]], cbuf, sem1)
            compute(j + 1, rbuf2, cbuf2)
            return carry
        lax.fori_loop(0, NCH // 2, chunk2, 0)
        pltpu.sync_copy(degl, deg_hbm.at[wid])

    return pl.kernel(body, out_type=out_type, mesh=_mesh,
                     compiler_params=pltpu.CompilerParams(
                         needs_layout_passes=False),
                     scratch_types=scratch)


def _make_agg_pass(D, CB):
    NCB = EPT // CB
    T = NCB // 3
    DH = D // 2
    out_type = jax.ShapeDtypeStruct((NC, NPAD, D), f32)
    scratch = [
        pltpu.VMEM((3, CB), i32),
        pltpu.VMEM((3, CB), i32),
        pltpu.VMEM((3, CB), i32),
        pltpu.VMEM((CB,), i32),
        pltpu.VMEM((CB,), i32),
        pltpu.VMEM((CB,), i32),
        pltpu.VMEM((CB, DH), i32),
        pltpu.VMEM((CB, DH), i32),
        pltpu.VMEM((CB, DH), i32),
        pltpu.VMEM((CB, D), f32),
        pltpu.VMEM((CB, D), f32),
        pltpu.VMEM((CB, D), f32),
        pltpu.VMEM((CB + 16,), f32),
        pltpu.VMEM_SHARED((NPAD, D), f32),
        pltpu.SemaphoreType.DMA,
        pltpu.SemaphoreType.DMA,
        pltpu.SemaphoreType.DMA,
    ]

    def body(feat_hbm, ivp_hbm, acc_hbm, iv0, iv1, iv2, rowb0, rowb1, rowb2,
             gi0, gi1, gi2, sbuf0, sbuf1, sbuf2, wbuf, acc_sh,
             semi, semg, sems):
        cid = lax.axis_index("c")
        sid = lax.axis_index("s")
        wid = sid * NC + cid

        zero16 = jnp.zeros((16,), f32)
        himask = jnp.full((16,), -65536, i32)

        def zrow(r, carry):
            for k in range(D // 16):
                sbuf0[r, pl.ds(k * 16, 16)] = zero16
            return carry
        lax.fori_loop(0, CB, zrow, 0)

        def zacc(t, carry):
            pltpu.sync_copy(sbuf0, acc_sh.at[pl.ds(sid * SL + t * CB, CB)])
            return carry
        lax.fori_loop(0, SL // CB, zacc, 0)
        plsc.subcore_barrier()

        def compute_scale(iv, rowb, gi, sbuf):
            for g in range(CB // 16):
                sl16 = pl.ds(g * 16, 16)
                wbuf[sl16] = plsc.bitcast(iv[2, sl16], f32)
                rowb[sl16] = iv[0, sl16]

            @plsc.parallel_loop(0, CB, 1, unroll=4)
            def scale(e):
                w16 = wbuf[pl.ds(e, 16)]
                wv = jnp.full((16,), w16[0], f32)
                for k in range(D // 32):
                    wi = gi[e, pl.ds(k * 16, 16)]
                    lo = plsc.bitcast(wi << 16, f32)
                    hi = plsc.bitcast(wi & himask, f32)
                    sbuf[e, pl.ds(k * 32, 16)] = lo * wv
                    sbuf[e, pl.ds(k * 32 + 16, 16)] = hi * wv

        def wait_iv(j, iv):
            pltpu.make_async_copy(ivp_hbm.at[wid, j], iv, semi).wait()

        def wait_gather(iv, gi):
            pltpu.make_async_copy(feat_hbm.at[iv.at[1]], gi, semg).wait()

        def wait_scatter(sbuf, rowb):
            pltpu.make_async_copy(sbuf, acc_sh.at[rowb], sems).wait()

        def triple(t, carry):
            j = 3 * t

            @pl.when(t == 0)
            def _():
                pltpu.sync_copy(ivp_hbm.at[wid, 0 * t], iv0)
                pltpu.async_copy(feat_hbm.at[iv0.at[1]], gi0, semg)
                pltpu.async_copy(ivp_hbm.at[wid, 0 * t + 1], iv1, semi)

            wait_gather(iv0, gi0)
            wait_iv(j + 1, iv1)

            @pl.when(t > 0)
            def _():
                wait_scatter(sbuf1, rowb1)
            pltpu.async_copy(feat_hbm.at[iv1.at[1]], gi1, semg)
            compute_scale(iv0, rowb0, gi0, sbuf0)
            pltpu.async_copy(sbuf0, acc_sh.at[rowb0], sems, add=True)
            pltpu.async_copy(ivp_hbm.at[wid, j + 2], iv2, semi)

            wait_gather(iv1, gi1)
            wait_iv(j + 2, iv2)

            @pl.when(t > 0)
            def _():
                wait_scatter(sbuf2, rowb2)
            pltpu.async_copy(feat_hbm.at[iv2.at[1]], gi2, semg)
            compute_scale(iv1, rowb1, gi1, sbuf1)
            pltpu.async_copy(sbuf1, acc_sh.at[rowb1], sems, add=True)

            @pl.when(t < T - 1)
            def _():
                pltpu.async_copy(ivp_hbm.at[wid, j + 3], iv0, semi)

            wait_gather(iv2, gi2)
            wait_scatter(sbuf0, rowb0)

            @pl.when(t < T - 1)
            def _():
                wait_iv(j + 3, iv0)
                pltpu.async_copy(feat_hbm.at[iv0.at[1]], gi0, semg)
                pltpu.async_copy(ivp_hbm.at[wid, j + 4], iv1, semi)
            compute_scale(iv2, rowb2, gi2, sbuf2)
            pltpu.async_copy(sbuf2, acc_sh.at[rowb2], sems, add=True)
            return carry
        lax.fori_loop(0, T, triple, 0)
        wait_scatter(sbuf1, rowb1)
        wait_scatter(sbuf2, rowb2)
        plsc.subcore_barrier()

        def dump(t, carry):
            pltpu.sync_copy(acc_sh.at[pl.ds(sid * SL + t * CB, CB)], sbuf0)
            pltpu.sync_copy(sbuf0, acc_hbm.at[cid, pl.ds(sid * SL + t * CB, CB)])
            return carry
        lax.fori_loop(0, SL // CB, dump, 0)

    return pl.kernel(body, out_type=out_type, mesh=_mesh,
                     compiler_params=pltpu.CompilerParams(
                         needs_layout_passes=False,
                         use_tc_tiling_on_sc=False),
                     scratch_types=scratch)


_sim_pass1 = _make_sim_pass(False)
_sim_pass2 = _make_sim_pass(True)
CB1 = 64
CB2 = 128
_agg_pass1 = _make_agg_pass(F, CB1)
_agg_pass2 = _make_agg_pass(NCLASS, CB2)


def _tc1_body(x_ref, xn_ref):
    x = x_ref[...]
    nr = jnp.sqrt(jnp.sum(x * x, axis=1, keepdims=True))
    xn_ref[...] = x / jnp.maximum(nr, 1e-8)


def _tc1(x):
    return pl.pallas_call(
        _tc1_body,
        out_shape=jax.ShapeDtypeStruct((N, F), f32),
    )(x)


def _pack_tc(z):
    u = lax.bitcast_convert_type(z.astype(jnp.bfloat16), jnp.uint16)
    words = []
    for k in range(z.shape[1] // 32):
        lo = u[:, k * 32:k * 32 + 16].astype(jnp.uint32)
        hi = u[:, k * 32 + 16:k * 32 + 32].astype(jnp.uint32)
        words.append(lo | (hi << 16))
    return lax.bitcast_convert_type(jnp.concatenate(words, axis=1), i32)


def _tc2_body(degp_ref, x_ref, dinv_ref, z1p_ref):
    deg = jnp.sum(degp_ref[...], axis=0)[:, None] + 1.0
    dinv = lax.rsqrt(deg)
    dinv_ref[...] = dinv
    z1p_ref[...] = _pack_tc(dinv[:N] * x_ref[...])


def _tc2(degp, x):
    return pl.pallas_call(
        _tc2_body,
        out_shape=(jax.ShapeDtypeStruct((NPAD, 1), f32),
                   jax.ShapeDtypeStruct((N, F // 2), i32)),
    )(degp, x)


def _tc3_body(accp_ref, x_ref, dinv_ref, W1_ref, b1_ref, hn_ref, nr2c_ref):
    dinv = dinv_ref[...][:N]
    acc = accp_ref[0, :N] + accp_ref[1, :N]
    pre = dinv * acc + (dinv * dinv) * x_ref[...]
    h = jnp.maximum(jnp.dot(pre, W1_ref[...],
                            preferred_element_type=f32) + b1_ref[...], 0.0)
    nr2 = jnp.sqrt(jnp.sum(h * h, axis=1, keepdims=True))
    nr2c = jnp.maximum(nr2, 1e-8)
    hn_ref[...] = h / nr2c
    nr2c_ref[...] = jnp.concatenate(
        [nr2c, jnp.ones((NPAD - N, 1), f32)], axis=0)


def _tc3(accp, x, dinv1, W1, b1):
    return pl.pallas_call(
        _tc3_body,
        out_shape=(jax.ShapeDtypeStruct((N, F), f32),
                   jax.ShapeDtypeStruct((NPAD, 1), f32)),
    )(accp, x, dinv1, W1, b1)


def _tc4_body(degp_ref, nr2c_ref, hn_ref, W2_ref, dinv_ref, z2_ref, z2p_ref):
    deg = jnp.sum(degp_ref[...], axis=0)[:, None] + 1.0
    dinv = lax.rsqrt(deg)
    dinv_ref[...] = dinv
    scale = (dinv * nr2c_ref[...])[:N]
    z2 = jnp.dot(scale * hn_ref[...], W2_ref[...],
                 preferred_element_type=f32)
    z2_ref[...] = z2
    z2p_ref[...] = _pack_tc(z2)


def _tc4(degp, nr2c, hn, W2):
    return pl.pallas_call(
        _tc4_body,
        out_shape=(jax.ShapeDtypeStruct((NPAD, 1), f32),
                   jax.ShapeDtypeStruct((N, NCLASS), f32),
                   jax.ShapeDtypeStruct((N, NCLASS // 2), i32)),
    )(degp, nr2c, hn, W2)


def _tc5_body(accp_ref, z2_ref, dinv_ref, b2_ref, out_ref):
    acc = accp_ref[0, :N] + accp_ref[1, :N] + z2_ref[...]
    out_ref[...] = dinv_ref[...][:N] * acc + b2_ref[...]


def _tc5(accp, z2, dinv2, b2):
    return pl.pallas_call(
        _tc5_body,
        out_shape=jax.ShapeDtypeStruct((N, NCLASS), f32),
    )(accp, z2, dinv2, b2)


def _make_ivp(rowf, colf, val, CB):
    NCB = EPT // CB
    r = rowf.reshape(NW, NCB, 1, CB)
    c = colf.reshape(NW, NCB, 1, CB)
    v = lax.bitcast_convert_type(val, i32).reshape(NW, NCB, 1, CB)
    return jnp.concatenate([r, c, v], axis=2)


def kernel(x, adj, W1, b1, W2, b2):
    E = adj.shape[1]
    pad = EPAD - E
    row = adj[0]
    col = adj[1]
    zpad = jnp.arange(pad, dtype=i32) % N
    rowf = jnp.concatenate([row, zpad])
    colf = jnp.concatenate([col, zpad])
    rowp = rowf.reshape(NW, NCH, C)
    colp = colf.reshape(NW, NCH, C)

    xn = _tc1(x)
    val1, deg1p = _sim_pass1(xn, rowp, colp)
    dinv1, z1p = _tc2(deg1p, x)
    acc1p = _agg_pass1(z1p, _make_ivp(rowf, colf, val1, CB1))
    hn, nr2c = _tc3(acc1p, x, dinv1, W1, b1)
    val2, deg2p = _sim_pass2(hn, rowp, colp, val1)
    dinv2, z2, z2p = _tc4(deg2p, nr2c, hn, W2)
    acc2p = _agg_pass2(z2p, _make_ivp(rowf, colf, val2, CB2))
    return _tc5(acc2p, z2, dinv2, b2)


# speedup vs baseline: 1.1797x; 1.0008x over previous
import jax
import jax.numpy as jnp
from jax import lax
from jax.experimental import pallas as pl
from jax.experimental.pallas import tpu as pltpu
from jax.experimental.pallas import tpu_sc as plsc

N = 10000
NPAD = 10240
F = 128
NCLASS = 64
THR = 0.1
NC = 2
NS = 16
NW = NC * NS
C = 128
EPT = 10752
NCH = EPT // C
EPAD = NW * EPT
SL = NPAD // NS
f32 = jnp.float32
i32 = jnp.int32

_mesh = plsc.VectorSubcoreMesh(core_axis_name="c", subcore_axis_name="s")


def _make_sim_pass(has_prev):
    out_type = (jax.ShapeDtypeStruct((NW, NCH, C), f32),
                jax.ShapeDtypeStruct((NW, NPAD), f32))
    scratch = [
        pltpu.VMEM((NCH, C), i32),
        pltpu.VMEM((NCH, C), i32),
        pltpu.VMEM((C, F), f32),
        pltpu.VMEM((C, F), f32),
        pltpu.VMEM((C, F), f32),
        pltpu.VMEM((C, F), f32),
        pltpu.VMEM((C,), f32),
        pltpu.VMEM((NPAD,), f32),
        pltpu.SemaphoreType.DMA,
        pltpu.SemaphoreType.DMA,
    ]
    if has_prev:
        scratch.insert(2, pltpu.VMEM((NCH, C), f32))

    def body(*refs):
        if has_prev:
            (feat_hbm, rowp_hbm, colp_hbm, prev_hbm, val_hbm, deg_hbm,
             rowv, colv, prevv, rbuf, cbuf, rbuf2, cbuf2, valb, degl,
             sem1, sem2) = refs
        else:
            (feat_hbm, rowp_hbm, colp_hbm, val_hbm, deg_hbm,
             rowv, colv, rbuf, cbuf, rbuf2, cbuf2, valb, degl,
             sem1, sem2) = refs
        cid = lax.axis_index("c")
        sid = lax.axis_index("s")
        wid = sid * NC + cid
        pltpu.sync_copy(rowp_hbm.at[wid], rowv)
        pltpu.sync_copy(colp_hbm.at[wid], colv)
        if has_prev:
            pltpu.sync_copy(prev_hbm.at[wid], prevv)

        zero16 = jnp.zeros((16,), f32)

        def zbody(i, carry):
            degl[pl.ds(i * 16, 16)] = zero16
            return carry
        lax.fori_loop(0, NPAD // 16, zbody, 0)

        iota = lax.iota(i32, 16)
        zv = jnp.zeros((16,), f32)
        z0 = jnp.zeros((16,), i32)

        def compute(j, rbuf_, cbuf_):
            for g in range(C // 16):
                rows16 = iota + (g * 16)

                def dot(i, accs):
                    a0, a1, a2, a3 = accs
                    kv = jnp.full((16,), i, i32) + iota
                    k0 = kv & (F - 1)
                    k1 = (kv + 1) & (F - 1)
                    k2 = (kv + 2) & (F - 1)
                    k3 = (kv + 3) & (F - 1)
                    x0 = plsc.load_gather(rbuf_, [rows16, k0])
                    y0 = plsc.load_gather(cbuf_, [rows16, k0])
                    x1 = plsc.load_gather(rbuf_, [rows16, k1])
                    y1 = plsc.load_gather(cbuf_, [rows16, k1])
                    x2 = plsc.load_gather(rbuf_, [rows16, k2])
                    y2 = plsc.load_gather(cbuf_, [rows16, k2])
                    x3 = plsc.load_gather(rbuf_, [rows16, k3])
                    y3 = plsc.load_gather(cbuf_, [rows16, k3])
                    return (a0 + x0 * y0, a1 + x1 * y1,
                            a2 + x2 * y2, a3 + x3 * y3)
                a0, a1, a2, a3 = plsc.parallel_loop(
                    0, F, 4, unroll=4, carry=(zv, zv, zv, zv))(dot)
                sims = (a0 + a1) + (a2 + a3)
                rv = rowv[j, pl.ds(g * 16, 16)]
                cv = colv[j, pl.ds(g * 16, 16)]
                m = (sims >= THR) & (rv != cv)
                if has_prev:
                    m = m & (prevv[j, pl.ds(g * 16, 16)] > 0.0)
                val = jnp.where(m, sims, 0.0)
                valb[pl.ds(g * 16, 16)] = val
                plsc.addupdate_scatter(degl, [rv], val)
            pltpu.sync_copy(valb, val_hbm.at[wid, j])

        def chunk2(jj, carry):
            j = jj * 2

            @pl.when(jj == 0)
            def _():
                pltpu.async_copy(feat_hbm.at[rowv.at[j]], rbuf, sem1)
                pltpu.async_copy(feat_hbm.at[colv.at[j]], cbuf, sem1)
            pltpu.make_async_copy(feat_hbm.at[rowv.at[j]], rbuf, sem1).wait()
            pltpu.make_async_copy(feat_hbm.at[colv.at[j]], cbuf, sem1).wait()
            pltpu.async_copy(feat_hbm.at[rowv.at[j + 1]], rbuf2, sem2)
            pltpu.async_copy(feat_hbm.at[colv.at[j + 1]], cbuf2, sem2)
            compute(j, rbuf, cbuf)
            pltpu.make_async_copy(
                feat_hbm.at[rowv.at[j + 1]], rbuf2, sem2).wait()
            pltpu.make_async_copy(
                feat_hbm.at[colv.at[j + 1]], cbuf2, sem2).wait()

            @pl.when(jj < NCH // 2 - 1)
            def _():
                pltpu.async_copy(feat_hbm.at[rowv.at[j + 2]], rbuf, sem1)
                pltpu.async_copy(feat_hbm.at[colv.at[j + 2]], cbuf, sem1)
            compute(j + 1, rbuf2, cbuf2)
            return carry
        lax.fori_loop(0, NCH // 2, chunk2, 0)
        pltpu.sync_copy(degl, deg_hbm.at[wid])

    return pl.kernel(body, out_type=out_type, mesh=_mesh,
                     compiler_params=pltpu.CompilerParams(
                         needs_layout_passes=False),
                     scratch_types=scratch)


def _make_agg_pass(D, CB):
    NCB = EPT // CB
    T = NCB // 3
    DH = D // 2
    out_type = jax.ShapeDtypeStruct((NC, NPAD, D), f32)
    scratch = [
        pltpu.VMEM((3, CB), i32),
        pltpu.VMEM((3, CB), i32),
        pltpu.VMEM((3, CB), i32),
        pltpu.VMEM((CB,), i32),
        pltpu.VMEM((CB,), i32),
        pltpu.VMEM((CB,), i32),
        pltpu.VMEM((CB, DH), i32),
        pltpu.VMEM((CB, DH), i32),
        pltpu.VMEM((CB, DH), i32),
        pltpu.VMEM((CB, D), f32),
        pltpu.VMEM((CB, D), f32),
        pltpu.VMEM((CB, D), f32),
        pltpu.VMEM((CB + 16,), f32),
        pltpu.VMEM_SHARED((NPAD, D), f32),
        pltpu.SemaphoreType.DMA,
        pltpu.SemaphoreType.DMA,
        pltpu.SemaphoreType.DMA,
    ]

    def body(feat_hbm, ivp_hbm, acc_hbm, iv0, iv1, iv2, rowb0, rowb1, rowb2,
             gi0, gi1, gi2, sbuf0, sbuf1, sbuf2, wbuf, acc_sh,
             semi, semg, sems):
        cid = lax.axis_index("c")
        sid = lax.axis_index("s")
        wid = sid * NC + cid

        zero16 = jnp.zeros((16,), f32)
        himask = jnp.full((16,), -65536, i32)

        def zrow(r, carry):
            for k in range(D // 16):
                sbuf0[r, pl.ds(k * 16, 16)] = zero16
            return carry
        lax.fori_loop(0, CB, zrow, 0)

        def zacc(t, carry):
            pltpu.sync_copy(sbuf0, acc_sh.at[pl.ds(sid * SL + t * CB, CB)])
            return carry
        lax.fori_loop(0, SL // CB, zacc, 0)
        plsc.subcore_barrier()

        def compute_scale(iv, rowb, gi, sbuf):
            for g in range(CB // 16):
                sl16 = pl.ds(g * 16, 16)
                wbuf[sl16] = plsc.bitcast(iv[2, sl16], f32)
                rowb[sl16] = iv[0, sl16]

            @plsc.parallel_loop(0, CB, 1, unroll=4)
            def scale(e):
                w16 = wbuf[pl.ds(e, 16)]
                wv = jnp.full((16,), w16[0], f32)
                for k in range(D // 32):
                    wi = gi[e, pl.ds(k * 16, 16)]
                    lo = plsc.bitcast(wi << 16, f32)
                    hi = plsc.bitcast(wi & himask, f32)
                    sbuf[e, pl.ds(k * 32, 16)] = lo * wv
                    sbuf[e, pl.ds(k * 32 + 16, 16)] = hi * wv

        def wait_iv(j, iv):
            pltpu.make_async_copy(ivp_hbm.at[wid, j], iv, semi).wait()

        def wait_gather(iv, gi):
            pltpu.make_async_copy(feat_hbm.at[iv.at[1]], gi, semg).wait()

        def wait_scatter(sbuf, rowb):
            pltpu.make_async_copy(sbuf, acc_sh.at[rowb], sems).wait()

        def triple(t, carry):
            j = 3 * t

            @pl.when(t == 0)
            def _():
                pltpu.sync_copy(ivp_hbm.at[wid, 0 * t], iv0)
                pltpu.async_copy(feat_hbm.at[iv0.at[1]], gi0, semg)
                pltpu.async_copy(ivp_hbm.at[wid, 0 * t + 1], iv1, semi)

            wait_gather(iv0, gi0)
            wait_iv(j + 1, iv1)

            @pl.when(t > 0)
            def _():
                wait_scatter(sbuf1, rowb1)
            pltpu.async_copy(feat_hbm.at[iv1.at[1]], gi1, semg)
            compute_scale(iv0, rowb0, gi0, sbuf0)
            pltpu.async_copy(sbuf0, acc_sh.at[rowb0], sems, add=True)
            pltpu.async_copy(ivp_hbm.at[wid, j + 2], iv2, semi)

            wait_gather(iv1, gi1)
            wait_iv(j + 2, iv2)

            @pl.when(t > 0)
            def _():
                wait_scatter(sbuf2, rowb2)
            pltpu.async_copy(feat_hbm.at[iv2.at[1]], gi2, semg)
            compute_scale(iv1, rowb1, gi1, sbuf1)
            pltpu.async_copy(sbuf1, acc_sh.at[rowb1], sems, add=True)

            @pl.when(t < T - 1)
            def _():
                pltpu.async_copy(ivp_hbm.at[wid, j + 3], iv0, semi)

            wait_gather(iv2, gi2)
            wait_scatter(sbuf0, rowb0)

            @pl.when(t < T - 1)
            def _():
                wait_iv(j + 3, iv0)
                pltpu.async_copy(feat_hbm.at[iv0.at[1]], gi0, semg)
                pltpu.async_copy(ivp_hbm.at[wid, j + 4], iv1, semi)
            compute_scale(iv2, rowb2, gi2, sbuf2)
            pltpu.async_copy(sbuf2, acc_sh.at[rowb2], sems, add=True)
            return carry
        lax.fori_loop(0, T, triple, 0)
        wait_scatter(sbuf1, rowb1)
        wait_scatter(sbuf2, rowb2)
        plsc.subcore_barrier()

        def dump(t, carry):
            pltpu.sync_copy(acc_sh.at[pl.ds(sid * SL + t * CB, CB)], sbuf0)
            pltpu.sync_copy(sbuf0, acc_hbm.at[cid, pl.ds(sid * SL + t * CB, CB)])
            return carry
        lax.fori_loop(0, SL // CB, dump, 0)

    return pl.kernel(body, out_type=out_type, mesh=_mesh,
                     compiler_params=pltpu.CompilerParams(
                         needs_layout_passes=False,
                         use_tc_tiling_on_sc=False),
                     scratch_types=scratch)


_sim_pass1 = _make_sim_pass(False)
_sim_pass2 = _make_sim_pass(True)
CB1 = 64
CB2 = 128
_agg_pass1 = _make_agg_pass(F, CB1)
_agg_pass2 = _make_agg_pass(NCLASS, CB2)


def _tc1_body(x_ref, xn_ref):
    x = x_ref[...]
    nr = jnp.sqrt(jnp.sum(x * x, axis=1, keepdims=True))
    xn_ref[...] = x / jnp.maximum(nr, 1e-8)


def _tc1(x):
    return pl.pallas_call(
        _tc1_body,
        out_shape=jax.ShapeDtypeStruct((N, F), f32),
    )(x)


def _pack_tc(z):
    u = lax.bitcast_convert_type(z.astype(jnp.bfloat16), jnp.uint16)
    words = []
    for k in range(z.shape[1] // 32):
        lo = u[:, k * 32:k * 32 + 16].astype(jnp.uint32)
        hi = u[:, k * 32 + 16:k * 32 + 32].astype(jnp.uint32)
        words.append(lo | (hi << 16))
    return lax.bitcast_convert_type(jnp.concatenate(words, axis=1), i32)


def _tc2_body(degp_ref, x_ref, dinv_ref, z1p_ref):
    deg = jnp.sum(degp_ref[...], axis=0)[:, None] + 1.0
    dinv = lax.rsqrt(deg)
    dinv_ref[...] = dinv
    z1p_ref[...] = _pack_tc(dinv[:N] * x_ref[...])


def _tc2(degp, x):
    return pl.pallas_call(
        _tc2_body,
        out_shape=(jax.ShapeDtypeStruct((NPAD, 1), f32),
                   jax.ShapeDtypeStruct((N, F // 2), i32)),
    )(degp, x)


def _tc3_body(accp_ref, x_ref, dinv_ref, W1_ref, b1_ref, hn_ref, nr2c_ref):
    dinv = dinv_ref[...][:N]
    acc = accp_ref[0, :N] + accp_ref[1, :N]
    pre = dinv * acc + (dinv * dinv) * x_ref[...]
    h = jnp.maximum(jnp.dot(pre, W1_ref[...],
                            preferred_element_type=f32) + b1_ref[...], 0.0)
    nr2 = jnp.sqrt(jnp.sum(h * h, axis=1, keepdims=True))
    nr2c = jnp.maximum(nr2, 1e-8)
    hn_ref[...] = h / nr2c
    nr2c_ref[...] = jnp.concatenate(
        [nr2c, jnp.ones((NPAD - N, 1), f32)], axis=0)


def _tc3(accp, x, dinv1, W1, b1):
    return pl.pallas_call(
        _tc3_body,
        out_shape=(jax.ShapeDtypeStruct((N, F), f32),
                   jax.ShapeDtypeStruct((NPAD, 1), f32)),
    )(accp, x, dinv1, W1, b1)


def _tc4_body(degp_ref, nr2c_ref, hn_ref, W2_ref, dinv_ref, z2_ref, z2p_ref):
    deg = jnp.sum(degp_ref[...], axis=0)[:, None] + 1.0
    dinv = lax.rsqrt(deg)
    dinv_ref[...] = dinv
    scale = (dinv * nr2c_ref[...])[:N]
    z2 = jnp.dot(scale * hn_ref[...], W2_ref[...],
                 preferred_element_type=f32)
    z2_ref[...] = z2
    z2p_ref[...] = _pack_tc(z2)


def _tc4(degp, nr2c, hn, W2):
    return pl.pallas_call(
        _tc4_body,
        out_shape=(jax.ShapeDtypeStruct((NPAD, 1), f32),
                   jax.ShapeDtypeStruct((N, NCLASS), f32),
                   jax.ShapeDtypeStruct((N, NCLASS // 2), i32)),
    )(degp, nr2c, hn, W2)


def _tc5_body(accp_ref, z2_ref, dinv_ref, b2_ref, out_ref):
    acc = accp_ref[0, :N] + accp_ref[1, :N] + z2_ref[...]
    out_ref[...] = dinv_ref[...][:N] * acc + b2_ref[...]


def _tc5(accp, z2, dinv2, b2):
    return pl.pallas_call(
        _tc5_body,
        out_shape=jax.ShapeDtypeStruct((N, NCLASS), f32),
    )(accp, z2, dinv2, b2)


def _make_ivp(rowf, colf, val, CB):
    NCB = EPT // CB
    r = rowf.reshape(NW, NCB, 1, CB)
    c = colf.reshape(NW, NCB, 1, CB)
    v = lax.bitcast_convert_type(val, i32).reshape(NW, NCB, 1, CB)
    return jnp.concatenate([r, c, v], axis=2)


def kernel(x, adj, W1, b1, W2, b2):
    E = adj.shape[1]
    pad = EPAD - E
    row = adj[0]
    col = adj[1]
    zpad = jnp.arange(pad, dtype=i32) % N
    rowf = jnp.concatenate([row, zpad])
    colf = jnp.concatenate([col, zpad])
    rowp = rowf.reshape(NW, NCH, C)
    colp = colf.reshape(NW, NCH, C)

    xn = _tc1(x)
    val1, deg1p = _sim_pass1(xn, rowp, colp)
    dinv1, z1p = _tc2(deg1p, x)
    acc1p = _agg_pass1(z1p, _make_ivp(rowf, colf, val1, CB1))
    hn, nr2c = _tc3(acc1p, x, dinv1, W1, b1)
    val2, deg2p = _sim_pass2(hn, rowp, colp, val1)
    dinv2, z2, z2p = _tc4(deg2p, nr2c, hn, W2)
    acc2p = _agg_pass2(z2p, _make_ivp(rowf, colf, val2, CB2))
    return _tc5(acc2p, z2, dinv2, b2)
